# baseline (speedup 1.0000x reference)
import functools

import jax
import jax.numpy as jnp
from jax import lax
from jax.experimental import pallas as pl
from jax.experimental.pallas import tpu as pltpu
from jax.experimental.pallas import tpu_sc as plsc

_V = 1000000
_D = 64
_SEQ = 200
_B = 4096
_NCLS = 10
_NP = 16
_NC = 2
_NS = 16
_NW = _NC * _NS
_BPW = _B // _NW
_LANES = 16
_NSLOT = 4

_CBLK = 32768
_PBLK = _CBLK // 8
_PSH = _PBLK.bit_length() - 1
_NBLK = (_V + _CBLK - 1) // _CBLK
_DECL = _NBLK * _CBLK

_mesh = plsc.VectorSubcoreMesh(core_axis_name="c", subcore_axis_name="s")


def _tc_proj(w_ref, t_ref, o_ref):
    p = jax.lax.dot_general(
        w_ref[...],
        t_ref[...],
        (((1,), (0,)), ((), ())),
        preferred_element_type=jnp.float32,
    )
    p8 = jnp.concatenate(
        [p[:, j * _PBLK : (j + 1) * _PBLK] for j in range(8)], axis=0
    )
    o_ref[...] = p8.T


@functools.partial(
    pl.kernel,
    out_type=jax.ShapeDtypeStruct((_B, _NP), jnp.float32),
    mesh=_mesh,
    scratch_types=[
        pltpu.VMEM((_SEQ, _BPW), jnp.int32),
        pltpu.VMEM((_NSLOT, _BPW, _NP), jnp.float32),
        pltpu.VMEM((_BPW, _NP), jnp.float32),
        pltpu.SemaphoreType.DMA,
        pltpu.SemaphoreType.DMA,
        pltpu.SemaphoreType.DMA,
        pltpu.SemaphoreType.DMA,
    ],
    compiler_params=pltpu.CompilerParams(use_tc_tiling_on_sc=False),
)
def _sc_pool(x_hbm, tab_hbm, out_hbm, idx_v, rows_v, pooled_v, s0, s1, s2, s3):
    wid = lax.axis_index("s") * _NC + lax.axis_index("c")
    base = wid * _BPW
    pltpu.sync_copy(x_hbm.at[:, pl.ds(base, _BPW)], idx_v)

    def xform_body(s, carry):
        for l in range(_BPW // _LANES):
            v = idx_v[s, pl.ds(l * _LANES, _LANES)]
            k = v & (_CBLK - 1)
            idx_v[s, pl.ds(l * _LANES, _LANES)] = (
                (v - k) + ((k & (_PBLK - 1)) << 3) + (k >> _PSH)
            )
        return carry

    lax.fori_loop(0, _SEQ, xform_body, 0)

    sems = (s0, s1, s2, s3)

    def start(r, slot):
        pltpu.async_copy(tab_hbm.at[idx_v.at[r]], rows_v.at[slot], sems[slot])

    def drain(slot):
        pltpu.make_async_copy(
            tab_hbm.at[pl.ds(0, _BPW)], rows_v.at[slot], sems[slot]
        ).wait()

    def zero_body(j, carry):
        pooled_v[j, pl.ds(0, _NP)] = jnp.zeros((_NP,), jnp.float32)
        return carry

    lax.fori_loop(0, _BPW, zero_body, 0)

    for slot in range(_NSLOT - 1):
        start(slot, slot)

    def accum(slot):
        def body(j, carry):
            plsc.addupdate(
                pooled_v.at[j, pl.ds(0, _NP)],
                rows_v[slot, j, pl.ds(0, _NP)],
            )
            return carry

        lax.fori_loop(0, _BPW, body, 0, unroll=16)

    def outer(g, carry):
        for k in range(_NSLOT):
            r = g * _NSLOT + k
            drain(k)

            @pl.when(r + _NSLOT - 1 < _SEQ)
            def _():
                start(r + _NSLOT - 1, (k + _NSLOT - 1) % _NSLOT)

            accum(k)
        return carry

    lax.fori_loop(0, _SEQ // _NSLOT, outer, 0)

    pltpu.sync_copy(pooled_v, out_hbm.at[pl.ds(base, _BPW)])


def _tc_head(p_ref, b_ref, o_ref):
    o_ref[...] = p_ref[...][:, 0:_NCLS] * (1.0 / _SEQ) + b_ref[...]


def kernel(x, table, W, b):
    wpad = jnp.zeros((_NP, _D), jnp.float32).at[0:_NCLS].set(W)
    lin = pl.pallas_call(
        _tc_proj,
        grid=(_NBLK,),
        in_specs=[
            pl.BlockSpec((_NP, _D), lambda i: (0, 0)),
            pl.BlockSpec((_D, _CBLK), lambda i: (0, i)),
        ],
        out_specs=pl.BlockSpec((_PBLK, 8 * _NP), lambda i: (i, 0)),
        out_shape=jax.ShapeDtypeStruct((_NBLK * _PBLK, 8 * _NP), jnp.float32),
    )(wpad, table.T)
    tabp = lin.reshape(_DECL, _NP)
    pooled = _sc_pool(x.astype(jnp.int32), tabp)
    return pl.pallas_call(
        _tc_head,
        out_shape=jax.ShapeDtypeStruct((_B, _NCLS), jnp.float32),
    )(pooled, b.reshape(1, _NCLS))


# speedup vs baseline: 1.0460x; 1.0460x over previous
import functools

import jax
import jax.numpy as jnp
from jax import lax
from jax.experimental import pallas as pl
from jax.experimental.pallas import tpu as pltpu
from jax.experimental.pallas import tpu_sc as plsc

_V = 1000000
_D = 64
_SEQ = 200
_B = 4096
_NCLS = 10
_NP = 16
_NC = 2
_NS = 16
_NW = _NC * _NS
_BPW = _B // _NW
_LANES = 16
_NSLOT = 4

_CBLK = 32768
_PBLK = _CBLK // 8
_PSH = _PBLK.bit_length() - 1
_NBLK = (_V + _CBLK - 1) // _CBLK
_DECL = _NBLK * _CBLK

_mesh = plsc.VectorSubcoreMesh(core_axis_name="c", subcore_axis_name="s")


def _tc_proj(w_ref, t_ref, o_ref):
    p = jax.lax.dot_general(
        w_ref[...],
        t_ref[...],
        (((1,), (0,)), ((), ())),
        preferred_element_type=jnp.float32,
    )
    p8 = jnp.concatenate(
        [p[:, j * _PBLK : (j + 1) * _PBLK] for j in range(8)], axis=0
    )
    o_ref[...] = p8.T


@functools.partial(
    pl.kernel,
    out_type=jax.ShapeDtypeStruct((_B, _NP), jnp.float32),
    mesh=_mesh,
    scratch_types=[
        pltpu.VMEM((_SEQ, _BPW), jnp.int32),
        pltpu.VMEM((_NSLOT, _BPW, _NP), jnp.float32),
        pltpu.VMEM((_BPW, _NP), jnp.float32),
        pltpu.SemaphoreType.DMA,
        pltpu.SemaphoreType.DMA,
        pltpu.SemaphoreType.DMA,
        pltpu.SemaphoreType.DMA,
    ],
    compiler_params=pltpu.CompilerParams(use_tc_tiling_on_sc=False),
)
def _sc_pool(x_hbm, tab_hbm, out_hbm, idx_v, rows_v, pooled_v, s0, s1, s2, s3):
    wid = lax.axis_index("s") * _NC + lax.axis_index("c")
    base = wid * _BPW
    pltpu.sync_copy(x_hbm.at[:, pl.ds(base, _BPW)], idx_v)

    def xform_body(s, carry):
        for l in range(_BPW // _LANES):
            v = idx_v[s, pl.ds(l * _LANES, _LANES)]
            k = v & (_CBLK - 1)
            idx_v[s, pl.ds(l * _LANES, _LANES)] = (
                (v - k) + ((k & (_PBLK - 1)) << 3) + (k >> _PSH)
            )
        return carry

    lax.fori_loop(0, _SEQ, xform_body, 0, unroll=2)

    sems = (s0, s1, s2, s3)

    def start(r, slot):
        pltpu.async_copy(tab_hbm.at[idx_v.at[r]], rows_v.at[slot], sems[slot])

    def drain(slot):
        pltpu.make_async_copy(
            tab_hbm.at[pl.ds(0, _BPW)], rows_v.at[slot], sems[slot]
        ).wait()

    def zero_body(j, carry):
        pooled_v[j, pl.ds(0, _NP)] = jnp.zeros((_NP,), jnp.float32)
        return carry

    lax.fori_loop(0, _BPW, zero_body, 0)

    for slot in range(_NSLOT - 1):
        start(slot, slot)

    def accum(slot):
        def body(j, carry):
            plsc.addupdate(
                pooled_v.at[j, pl.ds(0, _NP)],
                rows_v[slot, j, pl.ds(0, _NP)],
            )
            return carry

        lax.fori_loop(0, _BPW, body, 0, unroll=8)

    def outer(g, carry):
        for k in range(_NSLOT):
            r = g * _NSLOT + k
            drain(k)

            @pl.when(r + _NSLOT - 1 < _SEQ)
            def _():
                start(r + _NSLOT - 1, (k + _NSLOT - 1) % _NSLOT)

            accum(k)
        return carry

    lax.fori_loop(0, _SEQ // _NSLOT, outer, 0)

    pltpu.sync_copy(pooled_v, out_hbm.at[pl.ds(base, _BPW)])


def _tc_head(p_ref, b_ref, o_ref):
    o_ref[...] = p_ref[...][:, 0:_NCLS] * (1.0 / _SEQ) + b_ref[...]


def kernel(x, table, W, b):
    wpad = jnp.zeros((_NP, _D), jnp.float32).at[0:_NCLS].set(W)
    lin = pl.pallas_call(
        _tc_proj,
        grid=(_NBLK,),
        in_specs=[
            pl.BlockSpec((_NP, _D), lambda i: (0, 0)),
            pl.BlockSpec((_D, _CBLK), lambda i: (0, i)),
        ],
        out_specs=pl.BlockSpec((_PBLK, 8 * _NP), lambda i: (i, 0)),
        out_shape=jax.ShapeDtypeStruct((_NBLK * _PBLK, 8 * _NP), jnp.float32),
    )(wpad, table.T)
    tabp = lin.reshape(_DECL, _NP)
    pooled = _sc_pool(x.astype(jnp.int32), tabp)
    return pl.pallas_call(
        _tc_head,
        out_shape=jax.ShapeDtypeStruct((_B, _NCLS), jnp.float32),
    )(pooled, b.reshape(1, _NCLS))
